# baseline (device time: 61810 ns/iter reference)
import os

import jax
import jax.numpy as jnp
from jax import lax
from jax.experimental import pallas as pl
from jax.experimental.pallas import tpu as pltpu

_VARIANT = os.environ.get("KVAR", "full")

N_DEV = 4
B_BLK = 2
SQ = 128
D = 512
H = 8
DH = 64
ROWS = B_BLK * SQ


def kernel(x, Wq, Wo, K_ext, V_ext):
    my = lax.axis_index("i")


    def body(x_ref, wq_ref, wo_ref, k_hbm, v_hbm, out_ref,
             xg_ref, part_ref, rs_ref, q_ref, attn_ref, wq_b, wo_b,
             k_ref, v_ref,
             ag_send, ag_recv, rs_send, rs_recv, kv_sems):
        my_i = lax.axis_index("i")
        right = lax.rem(my_i + 1, N_DEV)
        left = lax.rem(my_i + N_DEV - 1, N_DEV)

        kv_copies = []
        for h in range(H):
            for src, dst, sem in ((k_hbm, k_ref, 0), (v_hbm, v_ref, 1)):
                c = pltpu.make_async_copy(
                    src.at[:, :, my_i * H + h, :], dst.at[h], kv_sems.at[sem]
                )
                c.start()
                kv_copies.append(c)

        barrier_sem = pltpu.get_barrier_semaphore()
        for nbr in (left, right):
            pl.semaphore_signal(
                barrier_sem, inc=1,
                device_id=(nbr,), device_id_type=pl.DeviceIdType.MESH,
            )
        pl.semaphore_wait(barrier_sem, 2)

        def compute_block(r):
            if _VARIANT == "nocompute":
                part_ref[r] = xg_ref[r]
                return
            origin = lax.rem(my_i + N_DEV - r, N_DEV)
            q_ref[:, :] = jnp.dot(
                xg_ref[r], wq_b[:, :], preferred_element_type=jnp.float32
            ).astype(jnp.bfloat16)
            if _VARIANT == "noattn":
                part_ref[r] = jnp.dot(
                    q_ref[:, :], wo_b[:, :],
                    preferred_element_type=jnp.float32,
                ).astype(jnp.bfloat16)
                return
            for b2 in range(B_BLK):
                bg = origin * B_BLK + b2
                for h in range(H):
                    qs = q_ref[b2 * SQ:(b2 + 1) * SQ, h * DH:(h + 1) * DH]
                    ks = k_ref[h, bg].astype(jnp.bfloat16)
                    s = lax.dot_general(
                        qs, ks, (((1,), (1,)), ((), ())),
                        preferred_element_type=jnp.float32,
                    ) * 0.125
                    m = jnp.max(s, axis=-1, keepdims=True)
                    p = jnp.exp(s - m)
                    l = jnp.sum(p, axis=-1, keepdims=True)
                    a = jnp.dot(
                        p.astype(jnp.bfloat16),
                        v_ref[h, bg].astype(jnp.bfloat16),
                        preferred_element_type=jnp.float32,
                    ) / l
                    attn_ref[b2 * SQ:(b2 + 1) * SQ, h * DH:(h + 1) * DH] = (
                        a.astype(jnp.bfloat16)
                    )
            part_ref[r] = jnp.dot(
                attn_ref[:, :], wo_b[:, :], preferred_element_type=jnp.float32
            ).astype(jnp.bfloat16)

        wq_b[:, :] = wq_ref[:, :].astype(jnp.bfloat16)
        wo_b[:, :] = wo_ref[:, :].astype(jnp.bfloat16)

        for c in kv_copies:
            c.wait()

        if _VARIANT == "nocomm":
            xg_ref[0] = x_ref[:, :, :].reshape(ROWS, D).astype(jnp.bfloat16)
            for r in range(N_DEV):
                compute_block(r)
            out_ref[:, :, :] = (
                part_ref[0].astype(jnp.float32)
                + part_ref[1].astype(jnp.float32)
                + part_ref[2].astype(jnp.float32)
                + part_ref[3].astype(jnp.float32)
            ).reshape(B_BLK, SQ, D)
            return

        xg_ref[0] = x_ref[:, :, :].reshape(ROWS, D).astype(jnp.bfloat16)

        def rs_send_block(r):
            rdma = pltpu.make_async_remote_copy(
                src_ref=part_ref.at[r],
                dst_ref=rs_ref.at[r - 1],
                send_sem=rs_send.at[r - 1],
                recv_sem=rs_recv.at[r - 1],
                device_id=(lax.rem(my_i + N_DEV - r, N_DEV),),
                device_id_type=pl.DeviceIdType.MESH,
            )
            rdma.start()
            return rdma

        rs_rdmas = []
        for hop in range(N_DEV - 1):
            ag = pltpu.make_async_remote_copy(
                src_ref=xg_ref.at[hop],
                dst_ref=xg_ref.at[hop + 1],
                send_sem=ag_send.at[hop],
                recv_sem=ag_recv.at[hop],
                device_id=(right,),
                device_id_type=pl.DeviceIdType.MESH,
            )
            ag.start()
            compute_block(hop)
            if hop > 0:
                rs_rdmas.append(rs_send_block(hop))
            ag.wait()
        compute_block(N_DEV - 1)
        rs_rdmas.append(rs_send_block(N_DEV - 1))

        def recv_desc(j):
            return pltpu.make_async_remote_copy(
                src_ref=part_ref.at[j + 1],
                dst_ref=rs_ref.at[j],
                send_sem=rs_send.at[j],
                recv_sem=rs_recv.at[j],
                device_id=(left,),
                device_id_type=pl.DeviceIdType.MESH,
            )

        recv_desc(0).wait_recv()
        recv_desc(1).wait_recv()
        acc = (
            part_ref[0].astype(jnp.float32)
            + rs_ref[0].astype(jnp.float32)
            + rs_ref[1].astype(jnp.float32)
        )
        recv_desc(2).wait_recv()
        out_ref[:, :, :] = (
            acc + rs_ref[2].astype(jnp.float32)
        ).reshape(B_BLK, SQ, D)

        for rdma in rs_rdmas:
            rdma.wait_send()

    return pl.pallas_call(
        body,
        out_shape=jax.ShapeDtypeStruct((B_BLK, SQ, D), jnp.float32),
        in_specs=[pl.BlockSpec(memory_space=pltpu.VMEM)] * 3
        + [pl.BlockSpec(memory_space=pl.ANY)] * 2,
        out_specs=pl.BlockSpec(memory_space=pltpu.VMEM),
        scratch_shapes=[
            pltpu.VMEM((N_DEV, ROWS, D), jnp.bfloat16),
            pltpu.VMEM((N_DEV, ROWS, D), jnp.bfloat16),
            pltpu.VMEM((N_DEV - 1, ROWS, D), jnp.bfloat16),
            pltpu.VMEM((ROWS, D), jnp.bfloat16),
            pltpu.VMEM((ROWS, D), jnp.bfloat16),
            pltpu.VMEM((D, D), jnp.bfloat16),
            pltpu.VMEM((D, D), jnp.bfloat16),
            pltpu.VMEM((H, N_DEV * B_BLK, SQ, DH), jnp.float32),
            pltpu.VMEM((H, N_DEV * B_BLK, SQ, DH), jnp.float32),
            pltpu.SemaphoreType.DMA((N_DEV - 1,)),
            pltpu.SemaphoreType.DMA((N_DEV - 1,)),
            pltpu.SemaphoreType.DMA((N_DEV - 1,)),
            pltpu.SemaphoreType.DMA((N_DEV - 1,)),
            pltpu.SemaphoreType.DMA((2,)),
        ],
        compiler_params=pltpu.CompilerParams(collective_id=0),
    )(x, Wq, Wo, K_ext, V_ext)


# device time: 30853 ns/iter; 2.0034x vs baseline; 2.0034x over previous
import os

import jax
import jax.numpy as jnp
from jax import lax
from jax.experimental import pallas as pl
from jax.experimental.pallas import tpu as pltpu

_VARIANT = os.environ.get("KVAR", "full")

N_DEV = 4
B_BLK = 2
SQ = 128
D = 512
H = 8
DH = 64
ROWS = B_BLK * SQ


def kernel(x, Wq, Wo, K_ext, V_ext):
    my = lax.axis_index("i")

    def prep(t):
        t = t.reshape(N_DEV * B_BLK, SQ, 4 * H * DH)
        t = lax.dynamic_slice_in_dim(t, my * (H * DH), H * DH, axis=2)
        return t.astype(jnp.bfloat16)

    K_sl = prep(K_ext)
    V_sl = prep(V_ext)

    def body(x_ref, wq_ref, wo_ref, k_ref, v_ref, out_ref,
             xg_ref, part_ref, rs_ref, q_ref, attn_ref, wq_b, wo_b,
             ag_send, ag_recv, rs_send, rs_recv):
        my_i = lax.axis_index("i")
        right = lax.rem(my_i + 1, N_DEV)
        left = lax.rem(my_i + N_DEV - 1, N_DEV)

        barrier_sem = pltpu.get_barrier_semaphore()
        for nbr in (left, right):
            pl.semaphore_signal(
                barrier_sem, inc=1,
                device_id=(nbr,), device_id_type=pl.DeviceIdType.MESH,
            )
        pl.semaphore_wait(barrier_sem, 2)

        def compute_block(r):
            if _VARIANT == "nocompute":
                part_ref[r] = xg_ref[r]
                return
            origin = lax.rem(my_i + N_DEV - r, N_DEV)
            q_ref[:, :] = jnp.dot(
                xg_ref[r], wq_b[:, :], preferred_element_type=jnp.float32
            ).astype(jnp.bfloat16)
            if _VARIANT == "noattn":
                part_ref[r] = jnp.dot(
                    q_ref[:, :], wo_b[:, :],
                    preferred_element_type=jnp.float32,
                ).astype(jnp.bfloat16)
                return
            for b2 in range(B_BLK):
                bg = origin * B_BLK + b2
                for h in range(H):
                    qs = q_ref[b2 * SQ:(b2 + 1) * SQ, h * DH:(h + 1) * DH]
                    ks = k_ref[bg, :, h * DH:(h + 1) * DH]
                    s = lax.dot_general(
                        qs, ks, (((1,), (1,)), ((), ())),
                        preferred_element_type=jnp.float32,
                    ) * 0.125
                    m = jnp.max(s, axis=-1, keepdims=True)
                    p = jnp.exp(s - m)
                    l = jnp.sum(p, axis=-1, keepdims=True)
                    a = jnp.dot(
                        p.astype(jnp.bfloat16),
                        v_ref[bg, :, h * DH:(h + 1) * DH],
                        preferred_element_type=jnp.float32,
                    ) / l
                    attn_ref[b2 * SQ:(b2 + 1) * SQ, h * DH:(h + 1) * DH] = (
                        a.astype(jnp.bfloat16)
                    )
            part_ref[r] = jnp.dot(
                attn_ref[:, :], wo_b[:, :], preferred_element_type=jnp.float32
            ).astype(jnp.bfloat16)

        wq_b[:, :] = wq_ref[:, :].astype(jnp.bfloat16)
        wo_b[:, :] = wo_ref[:, :].astype(jnp.bfloat16)

        if _VARIANT == "nocomm":
            xg_ref[0] = x_ref[:, :, :].reshape(ROWS, D).astype(jnp.bfloat16)
            for r in range(N_DEV):
                compute_block(r)
            out_ref[:, :, :] = (
                part_ref[0].astype(jnp.float32)
                + part_ref[1].astype(jnp.float32)
                + part_ref[2].astype(jnp.float32)
                + part_ref[3].astype(jnp.float32)
            ).reshape(B_BLK, SQ, D)
            return

        xg_ref[0] = x_ref[:, :, :].reshape(ROWS, D).astype(jnp.bfloat16)

        def rs_send_block(r):
            rdma = pltpu.make_async_remote_copy(
                src_ref=part_ref.at[r],
                dst_ref=rs_ref.at[r - 1],
                send_sem=rs_send.at[r - 1],
                recv_sem=rs_recv.at[r - 1],
                device_id=(lax.rem(my_i + N_DEV - r, N_DEV),),
                device_id_type=pl.DeviceIdType.MESH,
            )
            rdma.start()
            return rdma

        rs_rdmas = []
        for hop in range(N_DEV - 1):
            ag = pltpu.make_async_remote_copy(
                src_ref=xg_ref.at[hop],
                dst_ref=xg_ref.at[hop + 1],
                send_sem=ag_send.at[hop],
                recv_sem=ag_recv.at[hop],
                device_id=(right,),
                device_id_type=pl.DeviceIdType.MESH,
            )
            ag.start()
            compute_block(hop)
            if hop > 0:
                rs_rdmas.append(rs_send_block(hop))
            ag.wait()
        compute_block(N_DEV - 1)
        rs_rdmas.append(rs_send_block(N_DEV - 1))

        def recv_desc(j):
            return pltpu.make_async_remote_copy(
                src_ref=part_ref.at[j + 1],
                dst_ref=rs_ref.at[j],
                send_sem=rs_send.at[j],
                recv_sem=rs_recv.at[j],
                device_id=(left,),
                device_id_type=pl.DeviceIdType.MESH,
            )

        recv_desc(0).wait_recv()
        recv_desc(1).wait_recv()
        acc = (
            part_ref[0].astype(jnp.float32)
            + rs_ref[0].astype(jnp.float32)
            + rs_ref[1].astype(jnp.float32)
        )
        recv_desc(2).wait_recv()
        out_ref[:, :, :] = (
            acc + rs_ref[2].astype(jnp.float32)
        ).reshape(B_BLK, SQ, D)

        for rdma in rs_rdmas:
            rdma.wait_send()

    return pl.pallas_call(
        body,
        out_shape=jax.ShapeDtypeStruct((B_BLK, SQ, D), jnp.float32),
        in_specs=[pl.BlockSpec(memory_space=pltpu.VMEM)] * 5,
        out_specs=pl.BlockSpec(memory_space=pltpu.VMEM),
        scratch_shapes=[
            pltpu.VMEM((N_DEV, ROWS, D), jnp.bfloat16),
            pltpu.VMEM((N_DEV, ROWS, D), jnp.bfloat16),
            pltpu.VMEM((N_DEV - 1, ROWS, D), jnp.bfloat16),
            pltpu.VMEM((ROWS, D), jnp.bfloat16),
            pltpu.VMEM((ROWS, D), jnp.bfloat16),
            pltpu.VMEM((D, D), jnp.bfloat16),
            pltpu.VMEM((D, D), jnp.bfloat16),
            pltpu.SemaphoreType.DMA((N_DEV - 1,)),
            pltpu.SemaphoreType.DMA((N_DEV - 1,)),
            pltpu.SemaphoreType.DMA((N_DEV - 1,)),
            pltpu.SemaphoreType.DMA((N_DEV - 1,)),
        ],
        compiler_params=pltpu.CompilerParams(collective_id=0),
    )(x, Wq, Wo, K_sl, V_sl)
